# Initial kernel scaffold; baseline (speedup 1.0000x reference)
import jax
import jax.numpy as jnp
from jax.experimental import pallas as pl


def kernel(x, edge_index, W1, att_src1, att_dst1, bias1, W2, att_src2, att_dst2, bias2, W3, att_src3, att_dst3, bias3):
    raise NotImplementedError("write your pallas kernel here")


# speedup vs baseline: 26.7408x; 26.7408x over previous
import functools

import jax
import jax.numpy as jnp
from jax import lax
from jax.experimental import pallas as pl
from jax.experimental.pallas import tpu as pltpu
from jax.experimental.pallas import tpu_sc as plsc

NC = 2
NS = 16
NW = NC * NS
L = 16
CH = 128


def _proj_first(x_ref, w_ref, s_ref, d_ref, h_ref, hs_ref, hd_ref):
    h = jnp.dot(x_ref[...], w_ref[...], preferred_element_type=jnp.float32)
    h_ref[...] = h
    hs_ref[...] = jnp.dot(h, s_ref[...], preferred_element_type=jnp.float32)
    hd_ref[...] = jnp.dot(h, d_ref[...], preferred_element_type=jnp.float32)


def _proj_mid(acc_ref, den_ref, b_ref, w_ref, s_ref, d_ref,
              h_ref, hs_ref, hd_ref):
    den = jnp.sum(den_ref[...], axis=1)
    a = acc_ref[0] + acc_ref[1]
    xg = a * (1.0 / den)[:, None] + b_ref[...]
    xe = jnp.where(xg > 0, xg, jnp.exp(xg) - 1.0)
    h = jnp.dot(xe, w_ref[...], preferred_element_type=jnp.float32)
    h_ref[...] = h
    hs_ref[...] = jnp.dot(h, s_ref[...], preferred_element_type=jnp.float32)
    hd_ref[...] = jnp.dot(h, d_ref[...], preferred_element_type=jnp.float32)


def _pool(acc_ref, den_ref, b_ref, o_ref, *, n):
    den = jnp.sum(den_ref[...], axis=1)
    a = acc_ref[0] + acc_ref[1]
    xg = a * (1.0 / den)[:, None] + b_ref[...]
    xe = jnp.where(xg > 0, xg, jnp.exp(xg) - 1.0)
    s = jnp.sum(xe, axis=0, keepdims=True) * (1.0 / n)

    @pl.when(pl.program_id(0) == 0)
    def _():
        o_ref[...] = jnp.zeros_like(o_ref)

    o_ref[...] += s


def _edge_body(n, n_acc, d, e_true, cpw, h_hbm, as_hbm, ad_hbm, src_hbm,
               dst_hbm, acc_hbm, den_hbm, as_v, ad_v, den_v, src_v, dst_v,
               ex_v, rows_v, acc_sh, sem):
    cid = lax.axis_index("c")
    sid = lax.axis_index("s")
    wid = sid * NC + cid
    rows_per_tile = n_acc // NS

    pltpu.sync_copy(as_hbm, as_v)
    pltpu.sync_copy(ad_hbm, ad_v)

    z16 = jnp.zeros((L,), jnp.float32)

    def _zden(i, c):
        den_v[pl.ds(i * L, L)] = z16
        return c

    lax.fori_loop(0, n_acc // L, _zden, 0)

    def _zrows(i, c):
        for k in range(d // L):
            rows_v[i, pl.ds(k * L, L)] = z16
        return c

    lax.fori_loop(0, CH, _zrows, 0)

    slab = rows_per_tile // 5

    def _zacc(t, c):
        pltpu.sync_copy(rows_v.at[pl.ds(0, slab)],
                        acc_sh.at[pl.ds(sid * rows_per_tile + t * slab, slab)])
        return c

    lax.fori_loop(0, 5, _zacc, 0)
    plsc.subcore_barrier()

    neg = jnp.full((L,), -3e38, jnp.float32)

    def _mx(i, m):
        return (jnp.maximum(m[0], as_v[pl.ds(i * L, L)]),
                jnp.maximum(m[1], ad_v[pl.ds(i * L, L)]))

    ms, md = lax.fori_loop(0, n // L, _mx, (neg, neg))
    msv, mdv = ms[0], md[0]
    for lane in range(1, L):
        msv = jnp.maximum(msv, ms[lane])
        mdv = jnp.maximum(mdv, md[lane])
    mt = msv + mdv
    cbound = jnp.maximum(mt, 0.2 * mt)

    base = wid * cpw * CH

    def _chunk(j, c):
        off = base + j * CH
        pltpu.sync_copy(src_hbm.at[pl.ds(off, CH)], src_v)
        pltpu.sync_copy(dst_hbm.at[pl.ds(off, CH)], dst_v)
        cp = pltpu.async_copy(h_hbm.at[src_v], rows_v, sem)
        for i in range(CH // L):
            s16 = src_v[pl.ds(i * L, L)]
            d16 = dst_v[pl.ds(i * L, L)]
            al = (plsc.load_gather(as_v, [s16]) +
                  plsc.load_gather(ad_v, [d16]))
            al = jnp.maximum(al, 0.2 * al) - cbound
            ex = jnp.exp(al)
            eidx = off + i * L + lax.iota(jnp.int32, L)
            ex = jnp.where(eidx < e_true, ex, 0.0)
            ex_v[pl.ds(i * L, L)] = ex
            plsc.addupdate_scatter(den_v, [d16], ex)
        cp.wait()

        def _srow(g, cc):
            ex16 = ex_v[pl.ds(g * L, L)]
            for lane in range(L):
                e = g * L + lane
                coef = ex16[lane]
                for k in range(d // L):
                    rows_v[e, pl.ds(k * L, L)] = (
                        rows_v[e, pl.ds(k * L, L)] * coef)
            return cc

        lax.fori_loop(0, CH // L, _srow, 0)
        pltpu.sync_copy(rows_v, acc_sh.at[dst_v], add=True)
        return c

    lax.fori_loop(0, cpw, _chunk, 0)
    plsc.subcore_barrier()

    def _wacc(t, c):
        r0 = sid * rows_per_tile + t * slab
        pltpu.sync_copy(acc_sh.at[pl.ds(r0, slab)],
                        acc_hbm.at[cid, pl.ds(r0, slab)])
        return c

    lax.fori_loop(0, 5, _wacc, 0)
    pltpu.sync_copy(den_v, den_hbm.at[wid, 0])


@functools.lru_cache(maxsize=None)
def _build(n, d, e):
    e_true = e + n
    cpw = -(-e_true // (NW * CH))
    e_pad = NW * cpw * CH
    n_acc = -(-n // (NS * CH)) * NS * CH
    r = 1000
    grid = n // r
    f32 = jnp.float32

    proj_first = pl.pallas_call(
        _proj_first,
        grid=(grid,),
        in_specs=[
            pl.BlockSpec((r, d), lambda i: (i, 0)),
            pl.BlockSpec((d, d), lambda i: (0, 0)),
            pl.BlockSpec((d, 1), lambda i: (0, 0)),
            pl.BlockSpec((d, 1), lambda i: (0, 0)),
        ],
        out_specs=[
            pl.BlockSpec((r, d), lambda i: (i, 0)),
            pl.BlockSpec((r, 1), lambda i: (i, 0)),
            pl.BlockSpec((r, 1), lambda i: (i, 0)),
        ],
        out_shape=[
            jax.ShapeDtypeStruct((n, d), f32),
            jax.ShapeDtypeStruct((n, 1), f32),
            jax.ShapeDtypeStruct((n, 1), f32),
        ],
    )

    proj_mid = pl.pallas_call(
        _proj_mid,
        grid=(grid,),
        in_specs=[
            pl.BlockSpec((NC, r, d), lambda i: (0, i, 0)),
            pl.BlockSpec((r, NW), lambda i: (i, 0)),
            pl.BlockSpec((1, d), lambda i: (0, 0)),
            pl.BlockSpec((d, d), lambda i: (0, 0)),
            pl.BlockSpec((d, 1), lambda i: (0, 0)),
            pl.BlockSpec((d, 1), lambda i: (0, 0)),
        ],
        out_specs=[
            pl.BlockSpec((r, d), lambda i: (i, 0)),
            pl.BlockSpec((r, 1), lambda i: (i, 0)),
            pl.BlockSpec((r, 1), lambda i: (i, 0)),
        ],
        out_shape=[
            jax.ShapeDtypeStruct((n, d), f32),
            jax.ShapeDtypeStruct((n, 1), f32),
            jax.ShapeDtypeStruct((n, 1), f32),
        ],
    )

    pool = pl.pallas_call(
        functools.partial(_pool, n=n),
        grid=(grid,),
        in_specs=[
            pl.BlockSpec((NC, r, d), lambda i: (0, i, 0)),
            pl.BlockSpec((r, NW), lambda i: (i, 0)),
            pl.BlockSpec((1, d), lambda i: (0, 0)),
        ],
        out_specs=pl.BlockSpec((1, d), lambda i: (0, 0)),
        out_shape=jax.ShapeDtypeStruct((1, d), f32),
    )

    mesh = plsc.VectorSubcoreMesh(core_axis_name="c", subcore_axis_name="s")
    edge_call = pl.kernel(
        functools.partial(_edge_body, n, n_acc, d, e_true, cpw),
        out_type=[
            jax.ShapeDtypeStruct((NC, n_acc, d), f32),
            jax.ShapeDtypeStruct((NW, 1, n_acc), f32),
        ],
        mesh=mesh,
        scratch_types=[
            pltpu.VMEM((n,), f32),
            pltpu.VMEM((n,), f32),
            pltpu.VMEM((n_acc,), f32),
            pltpu.VMEM((CH,), jnp.int32),
            pltpu.VMEM((CH,), jnp.int32),
            pltpu.VMEM((CH,), f32),
            pltpu.VMEM((CH, d), f32),
            pltpu.VMEM_SHARED((n_acc, d), f32),
            pltpu.SemaphoreType.DMA,
        ],
        compiler_params=pltpu.CompilerParams(needs_layout_passes=False),
    )
    return proj_first, proj_mid, pool, edge_call, e_pad, e_true, n_acc


def kernel(x, edge_index, W1, att_src1, att_dst1, bias1,
           W2, att_src2, att_dst2, bias2,
           W3, att_src3, att_dst3, bias3):
    n, d = x.shape
    e = edge_index.shape[1]
    (proj_first, proj_mid, pool, edge_call,
     e_pad, e_true, n_acc) = _build(n, d, e)

    loop = jnp.arange(n, dtype=jnp.int32)
    pad = jnp.zeros((e_pad - e_true,), jnp.int32)
    src = jnp.concatenate([edge_index[0], loop, pad])
    dst = jnp.concatenate([edge_index[1], loop, pad])

    h, hs, hd = proj_first(x, W1, att_src1.reshape(d, 1),
                           att_dst1.reshape(d, 1))
    acc, den = edge_call(h, hs.reshape(n), hd.reshape(n), src, dst)

    h, hs, hd = proj_mid(acc, den.reshape(NW, n_acc).T, bias1.reshape(1, d),
                         W2, att_src2.reshape(d, 1), att_dst2.reshape(d, 1))
    acc, den = edge_call(h, hs.reshape(n), hd.reshape(n), src, dst)

    h, hs, hd = proj_mid(acc, den.reshape(NW, n_acc).T, bias2.reshape(1, d),
                         W3, att_src3.reshape(d, 1), att_dst3.reshape(d, 1))
    acc, den = edge_call(h, hs.reshape(n), hd.reshape(n), src, dst)

    return pool(acc, den.reshape(NW, n_acc).T, bias3.reshape(1, d))
